# baseline (speedup 1.0000x reference)
import dataclasses
import functools

import jax
import jax.numpy as jnp
from jax import lax
from jax.experimental import pallas as pl
from jax.experimental.pallas import tpu as pltpu
from jax.experimental.pallas import tpu_sc as plsc

EMBED = 32
NUM_CORES = 2
NUM_SUBCORES = 16
NUM_WORKERS = NUM_CORES * NUM_SUBCORES
LANES = 16
IDX_CHUNK = 128

ROWS = 1000000
TILE_COLS = (ROWS + 127) // 128
GROUP_STRIDE = TILE_COLS * 1024


@functools.partial(jax.jit, static_argnames=("batch",))
def _gmf_sc(user2d, item2d, wb, utab_t, itab_t, batch):
    b_per_w = batch // NUM_WORKERS
    n_uc = b_per_w // IDX_CHUNK
    mesh = plsc.VectorSubcoreMesh(core_axis_name="c", subcore_axis_name="s")

    cp = dataclasses.replace(
        pltpu.CompilerParams(),
        needs_layout_passes=False,
        use_tc_tiling_on_sc=False,
    )

    @functools.partial(
        pl.kernel,
        mesh=mesh,
        out_type=jax.ShapeDtypeStruct((batch,), jnp.float32),
        scratch_types=[
            pltpu.VMEM((n_uc, IDX_CHUNK), jnp.int32),
            pltpu.VMEM((n_uc, IDX_CHUNK), jnp.int32),
            pltpu.VMEM((EMBED * n_uc, IDX_CHUNK), jnp.int32),
            pltpu.VMEM((EMBED * n_uc, IDX_CHUNK), jnp.int32),
            pltpu.VMEM((EMBED, b_per_w), jnp.float32),
            pltpu.VMEM((EMBED, b_per_w), jnp.float32),
            pltpu.VMEM((64,), jnp.float32),
            pltpu.VMEM((b_per_w,), jnp.float32),
            pltpu.SemaphoreType.DMA,
            pltpu.SemaphoreType.DMA,
        ],
        compiler_params=cp,
    )
    def k(user_hbm, item_hbm, wb_hbm, utab_hbm, itab_hbm, out_hbm,
          uid_v, iid_v, uix_v, iix_v, urows_v, irows_v, wb_v, out_v,
          sem_u, sem_i):
        wid = lax.axis_index("s") * NUM_CORES + lax.axis_index("c")
        base = wid * b_per_w

        pltpu.sync_copy(user_hbm.at[pl.ds(wid * n_uc, n_uc)], uid_v)
        pltpu.sync_copy(item_hbm.at[pl.ds(wid * n_uc, n_uc)], iid_v)
        pltpu.sync_copy(wb_hbm, wb_v)

        utab_flat = utab_hbm.at[0]
        itab_flat = itab_hbm.at[0]

        @pl.loop(0, EMBED)
        def _(d):
            kd = d * ROWS
            for c in range(n_uc):
                row = d * n_uc + c
                for kk in range(IDX_CHUNK // LANES):
                    sl = pl.ds(kk * LANES, LANES)
                    uix_v[row, sl] = uid_v[c, sl] + kd
                    iix_v[row, sl] = iid_v[c, sl] + kd
                dst = pl.ds(c * IDX_CHUNK, IDX_CHUNK)
                pltpu.async_copy(
                    utab_flat.at[uix_v.at[row]], urows_v.at[d, dst], sem_u)
                pltpu.async_copy(
                    itab_flat.at[iix_v.at[row]], irows_v.at[d, dst], sem_i)

        pltpu.make_async_copy(
            utab_hbm.at[:, pl.ds(0, b_per_w)], urows_v, sem_u).wait()
        pltpu.make_async_copy(
            itab_hbm.at[:, pl.ds(0, b_per_w)], irows_v, sem_i).wait()

        w_splat = [
            plsc.load_gather(wb_v, [jnp.full((LANES,), LANES + d, jnp.int32)])
            for d in range(EMBED)
        ]
        b_splat = plsc.load_gather(
            wb_v, [jnp.full((LANES,), LANES + EMBED, jnp.int32)])

        @pl.loop(0, b_per_w, step=LANES)
        def _(u0):
            sl = pl.ds(u0, LANES)
            acc = b_splat
            for d in range(EMBED):
                acc = acc + urows_v[d, sl] * irows_v[d, sl] * w_splat[d]
            out_v[sl] = acc

        pltpu.sync_copy(out_v, out_hbm.at[pl.ds(base, b_per_w)])

    return k(user2d, item2d, wb, utab_t, itab_t)


def kernel(user, item, user_table, item_table, W, b):
    batch = user.shape[0]
    user2d = user.astype(jnp.int32).reshape(-1, IDX_CHUNK)
    item2d = item.astype(jnp.int32).reshape(-1, IDX_CHUNK)
    wb = jnp.concatenate(
        [jnp.zeros((LANES,), jnp.float32),
         W.reshape(-1).astype(jnp.float32), b.astype(jnp.float32),
         jnp.zeros((64 - LANES - EMBED - 1,), jnp.float32)])
    return _gmf_sc(user2d, item2d, wb, user_table.T, item_table.T, batch)


# speedup vs baseline: 4.7991x; 4.7991x over previous
import dataclasses
import functools

import jax
import jax.numpy as jnp
from jax import lax
from jax.experimental import pallas as pl
from jax.experimental.pallas import tpu as pltpu
from jax.experimental.pallas import tpu_sc as plsc

EMBED = 32
NUM_CORES = 2
NUM_SUBCORES = 16
NUM_WORKERS = NUM_CORES * NUM_SUBCORES
LANES = 16
IDX_CHUNK = 128


@functools.partial(jax.jit, static_argnames=("batch",))
def _gmf_sc(user2d, item2d, wb, user_table, item_table, batch):
    b_per_w = batch // NUM_WORKERS
    n_chunk = b_per_w // IDX_CHUNK
    mesh = plsc.VectorSubcoreMesh(core_axis_name="c", subcore_axis_name="s")

    cp = dataclasses.replace(
        pltpu.CompilerParams(),
        use_tc_tiling_on_sc=False,
        needs_layout_passes=False,
    )

    @functools.partial(
        pl.kernel,
        mesh=mesh,
        out_type=jax.ShapeDtypeStruct((batch,), jnp.float32),
        scratch_types=[
            pltpu.VMEM((n_chunk, IDX_CHUNK), jnp.int32),
            pltpu.VMEM((n_chunk, IDX_CHUNK), jnp.int32),
            pltpu.VMEM((b_per_w, EMBED), jnp.bfloat16),
            pltpu.VMEM((b_per_w, EMBED), jnp.bfloat16),
            pltpu.VMEM((b_per_w, EMBED), jnp.float32),
            pltpu.VMEM((b_per_w, EMBED), jnp.float32),
            pltpu.VMEM((64,), jnp.float32),
            pltpu.VMEM((b_per_w,), jnp.float32),
            pltpu.SemaphoreType.DMA,
            pltpu.SemaphoreType.DMA,
        ],
        compiler_params=cp,
    )
    def k(user_hbm, item_hbm, wb_hbm, utab_hbm, itab_hbm, out_hbm,
          uidx_v, iidx_v, u16_v, i16_v, urows_v, irows_v, wb_v, out_v,
          sem_u, sem_i):
        wid = lax.axis_index("s") * NUM_CORES + lax.axis_index("c")
        base = wid * b_per_w

        pltpu.sync_copy(user_hbm.at[pl.ds(wid * n_chunk, n_chunk)], uidx_v)
        pltpu.sync_copy(item_hbm.at[pl.ds(wid * n_chunk, n_chunk)], iidx_v)
        pltpu.sync_copy(wb_hbm, wb_v)

        copies = []
        for c in range(n_chunk):
            copies.append(pltpu.async_copy(
                utab_hbm.at[uidx_v.at[c]],
                u16_v.at[pl.ds(c * IDX_CHUNK, IDX_CHUNK)], sem_u))
            copies.append(pltpu.async_copy(
                itab_hbm.at[iidx_v.at[c]],
                i16_v.at[pl.ds(c * IDX_CHUNK, IDX_CHUNK)], sem_i))
        for cp_ in copies:
            cp_.wait()

        @pl.loop(0, b_per_w)
        def _(r):
            ua, ub = plsc.unpack(
                u16_v[r, :], format=plsc.PackFormat.INTERLEAVED)
            urows_v[r, pl.ds(0, LANES)] = ua
            urows_v[r, pl.ds(LANES, LANES)] = ub
            ia, ib = plsc.unpack(
                i16_v[r, :], format=plsc.PackFormat.INTERLEAVED)
            irows_v[r, pl.ds(0, LANES)] = ia
            irows_v[r, pl.ds(LANES, LANES)] = ib

        w_splat = [
            plsc.load_gather(wb_v, [jnp.full((LANES,), LANES + d, jnp.int32)])
            for d in range(EMBED)
        ]
        b_splat = plsc.load_gather(
            wb_v, [jnp.full((LANES,), LANES + EMBED, jnp.int32)])
        lane_iota = lax.iota(jnp.int32, LANES)

        @pl.loop(0, b_per_w, step=LANES)
        def _(r0):
            row_idx = lane_iota + r0
            acc = b_splat
            for d in range(EMBED):
                d_idx = jnp.full((LANES,), d, jnp.int32)
                ug = plsc.load_gather(urows_v, [row_idx, d_idx])
                ig = plsc.load_gather(irows_v, [row_idx, d_idx])
                acc = acc + ug * ig * w_splat[d]
            out_v[pl.ds(r0, LANES)] = acc

        pltpu.sync_copy(out_v, out_hbm.at[pl.ds(base, b_per_w)])

    return k(user2d, item2d, wb, user_table, item_table)


def kernel(user, item, user_table, item_table, W, b):
    batch = user.shape[0]
    user2d = user.astype(jnp.int32).reshape(-1, IDX_CHUNK)
    item2d = item.astype(jnp.int32).reshape(-1, IDX_CHUNK)
    w_flat = W.reshape(-1).astype(jnp.float32)
    w_perm = jnp.concatenate([w_flat[0::2], w_flat[1::2]])
    wb = jnp.concatenate(
        [jnp.zeros((LANES,), jnp.float32),
         w_perm, b.astype(jnp.float32),
         jnp.zeros((64 - LANES - EMBED - 1,), jnp.float32)])
    return _gmf_sc(user2d, item2d, wb,
                   user_table.astype(jnp.bfloat16),
                   item_table.astype(jnp.bfloat16), batch)


# speedup vs baseline: 5.6344x; 1.1740x over previous
import dataclasses
import functools

import jax
import jax.numpy as jnp
from jax import lax
from jax.experimental import pallas as pl
from jax.experimental.pallas import tpu as pltpu
from jax.experimental.pallas import tpu_sc as plsc

EMBED = 32
NUM_CORES = 2
NUM_SUBCORES = 16
NUM_WORKERS = NUM_CORES * NUM_SUBCORES
LANES = 16
IDX_CHUNK = 128


@functools.partial(jax.jit, static_argnames=("batch",))
def _gmf_sc(user2d, item2d, wb, user_table, item_table, batch):
    b_per_w = batch // NUM_WORKERS
    n_chunk = b_per_w // IDX_CHUNK
    mesh = plsc.VectorSubcoreMesh(core_axis_name="c", subcore_axis_name="s")

    cp = dataclasses.replace(
        pltpu.CompilerParams(),
        use_tc_tiling_on_sc=False,
        needs_layout_passes=False,
    )

    @functools.partial(
        pl.kernel,
        mesh=mesh,
        out_type=jax.ShapeDtypeStruct((batch,), jnp.float32),
        scratch_types=[
            pltpu.VMEM((n_chunk, IDX_CHUNK), jnp.int32),
            pltpu.VMEM((n_chunk, IDX_CHUNK), jnp.int32),
            pltpu.VMEM((b_per_w, EMBED), jnp.float32),
            pltpu.VMEM((b_per_w, EMBED), jnp.float32),
            pltpu.VMEM((64,), jnp.float32),
            pltpu.VMEM((b_per_w,), jnp.float32),
            pltpu.SemaphoreType.DMA,
            pltpu.SemaphoreType.DMA,
        ],
        compiler_params=cp,
    )
    def k(user_hbm, item_hbm, wb_hbm, utab_hbm, itab_hbm, out_hbm,
          uidx_v, iidx_v, urows_v, irows_v, wb_v, out_v, sem_u, sem_i):
        wid = lax.axis_index("s") * NUM_CORES + lax.axis_index("c")
        base = wid * b_per_w

        pltpu.sync_copy(user_hbm.at[pl.ds(wid * n_chunk, n_chunk)], uidx_v)
        pltpu.sync_copy(item_hbm.at[pl.ds(wid * n_chunk, n_chunk)], iidx_v)
        pltpu.sync_copy(wb_hbm, wb_v)

        copies = []
        for c in range(n_chunk):
            copies.append(pltpu.async_copy(
                utab_hbm.at[uidx_v.at[c]],
                urows_v.at[pl.ds(c * IDX_CHUNK, IDX_CHUNK)], sem_u))
            copies.append(pltpu.async_copy(
                itab_hbm.at[iidx_v.at[c]],
                irows_v.at[pl.ds(c * IDX_CHUNK, IDX_CHUNK)], sem_i))
        for cp_ in copies:
            cp_.wait()

        w_splat = [
            plsc.load_gather(wb_v, [jnp.full((LANES,), LANES + d, jnp.int32)])
            for d in range(EMBED)
        ]
        b_splat = plsc.load_gather(
            wb_v, [jnp.full((LANES,), LANES + EMBED, jnp.int32)])
        lane_iota = lax.iota(jnp.int32, LANES)

        @pl.loop(0, b_per_w, step=LANES)
        def _(r0):
            row_idx = lane_iota + r0
            acc = b_splat
            for d in range(EMBED):
                d_idx = jnp.full((LANES,), d, jnp.int32)
                ug = plsc.load_gather(urows_v, [row_idx, d_idx])
                ig = plsc.load_gather(irows_v, [row_idx, d_idx])
                acc = acc + ug * ig * w_splat[d]
            out_v[pl.ds(r0, LANES)] = acc

        pltpu.sync_copy(out_v, out_hbm.at[pl.ds(base, b_per_w)])

    return k(user2d, item2d, wb, user_table, item_table)


def kernel(user, item, user_table, item_table, W, b):
    batch = user.shape[0]
    user2d = user.astype(jnp.int32).reshape(-1, IDX_CHUNK)
    item2d = item.astype(jnp.int32).reshape(-1, IDX_CHUNK)
    wb = jnp.concatenate(
        [jnp.zeros((LANES,), jnp.float32),
         W.reshape(-1).astype(jnp.float32), b.astype(jnp.float32),
         jnp.zeros((64 - LANES - EMBED - 1,), jnp.float32)])
    return _gmf_sc(user2d, item2d, wb, user_table, item_table, batch)
